# baseline (speedup 1.0000x reference)
import jax
import jax.numpy as jnp
from jax.experimental import pallas as pl


def kernel(inputs, indices, output_shape):
    Bs, Hp, Wp, Cs = inputs.shape
    ph, pw = 2, 2
    B, H, W, C = Bs, Hp * ph, Wp * pw, Cs
    flat_inputs = inputs.reshape(-1)
    flat_indices = indices.reshape(-1).astype(jnp.int32)
    total_elements = B * H * W * C
    per_batch = flat_indices.shape[0] // B
    batch_offset = jnp.repeat(jnp.arange(B, dtype=jnp.int32) * (H * W * C), per_batch)
    flat_indices = flat_indices + batch_offset
    output = jnp.zeros((total_elements,), dtype=flat_inputs.dtype)
    output = output.at[flat_indices].add(flat_inputs)
    return output.reshape(B, H, W, C)


# speedup vs baseline: 11.7326x; 11.7326x over previous
import functools

import jax
import jax.numpy as jnp
from jax import lax
from jax.experimental import pallas as pl
from jax.experimental.pallas import tpu as pltpu
from jax.experimental.pallas import tpu_sc as plsc

_NS = 16
_FB = 128
_UN = 8


@functools.lru_cache(maxsize=None)
def _build(B, NB, OUTB):
    ET = NB // _NS
    K = 6144
    assert ET % K == 0
    NCHUNK = ET // K
    NPASS = 12
    assert OUTB % NPASS == 0
    W = OUTB // NPASS
    assert W % (_NS * 16) == 0
    PERT = W // _NS
    ZB = 3584
    assert PERT % ZB == 0
    CAP = K + 2 * _FB

    mesh = plsc.VectorSubcoreMesh(core_axis_name="c", subcore_axis_name="s")

    @functools.partial(
        pl.kernel,
        out_type=jax.ShapeDtypeStruct((B * OUTB,), jnp.float32),
        mesh=mesh,
        compiler_params=pltpu.CompilerParams(needs_layout_passes=False),
        scratch_types=[
            pltpu.VMEM((K,), jnp.int32),
            pltpu.VMEM((K,), jnp.float32),
            pltpu.VMEM((CAP + 16,), jnp.int32),
            pltpu.VMEM((CAP + 16,), jnp.float32),
            pltpu.VMEM((ZB,), jnp.float32),
            pltpu.VMEM_SHARED((W,), jnp.float32),
        ],
    )
    def scatter_add(val_hbm, idx_hbm, out_hbm, idx_in, val_in, cidx, cval,
                    zbuf, win):
        c = lax.axis_index("c")
        s = lax.axis_index("s")
        base_elem = c * NB + s * ET
        zeros_f = jnp.zeros((16,), jnp.float32)
        zeros_i = jnp.zeros((16,), jnp.int32)
        ones_i = jnp.ones((16,), jnp.int32)

        def zb_body(i, _):
            zbuf[pl.ds(i * 16, 16)] = zeros_f
            return 0
        lax.fori_loop(0, ZB // 16, zb_body, 0)

        def pass_body(p, _):
            base = p * W

            def zw_body(z, _):
                pltpu.sync_copy(zbuf, win.at[pl.ds(s * PERT + z * ZB, ZB)])
                return 0
            lax.fori_loop(0, PERT // ZB, zw_body, 0)
            plsc.subcore_barrier()

            def chunk_body(j, fill):
                off = base_elem + j * K
                pltpu.sync_copy(idx_hbm.at[pl.ds(off, K)], idx_in)
                pltpu.sync_copy(val_hbm.at[pl.ds(off, K)], val_in)

                def vec_body(i, fill):
                    dat = []
                    for q in range(_UN):
                        o = (i * _UN + q) * 16
                        iv = idx_in[pl.ds(o, 16)]
                        vv = val_in[pl.ds(o, 16)]
                        t = iv - base
                        m = (t >= 0) & (t < W)
                        cnt = jnp.sum(jnp.where(m, ones_i, zeros_i))
                        dat.append((m, t, vv, cnt))
                    for q in range(_UN):
                        m, t, vv, cnt = dat[q]
                        plsc.store_compressed(cidx.at[pl.ds(fill, 16)], t,
                                              mask=m)
                        plsc.store_compressed(cval.at[pl.ds(fill, 16)], vv,
                                              mask=m)
                        fill = fill + cnt
                    return fill
                fill = lax.fori_loop(0, K // (16 * _UN), vec_body, fill)

                nblk = fill // _FB

                def fl_body(k, _):
                    pltpu.sync_copy(cval.at[pl.ds(k * _FB, _FB)],
                                    win.at[cidx.at[pl.ds(k * _FB, _FB)]],
                                    add=True)
                    return 0
                lax.fori_loop(0, nblk, fl_body, 0)

                rem = fill - nblk * _FB
                srcoff = nblk * _FB

                def mv_body(i, _):
                    cidx[pl.ds(i * 16, 16)] = cidx[pl.ds(srcoff + i * 16, 16)]
                    cval[pl.ds(i * 16, 16)] = cval[pl.ds(srcoff + i * 16, 16)]
                    return 0
                lax.fori_loop(0, (rem + 15) // 16, mv_body, 0)
                return rem
            fill = lax.fori_loop(0, NCHUNK, chunk_body, jnp.int32(0))

            def pad_body(i, _):
                cidx[pl.ds(fill + i * 16, 16)] = zeros_i
                cval[pl.ds(fill + i * 16, 16)] = zeros_f
                return 0
            lax.fori_loop(0, _FB // 16, pad_body, 0)
            nblk2 = (fill + _FB - 1) // _FB

            def fl2_body(k, _):
                pltpu.sync_copy(cval.at[pl.ds(k * _FB, _FB)],
                                win.at[cidx.at[pl.ds(k * _FB, _FB)]],
                                add=True)
                return 0
            lax.fori_loop(0, nblk2, fl2_body, 0)
            plsc.subcore_barrier()

            pltpu.sync_copy(
                win.at[pl.ds(s * PERT, PERT)],
                out_hbm.at[pl.ds(c * OUTB + base + s * PERT, PERT)])
            plsc.subcore_barrier()
            return 0
        lax.fori_loop(0, NPASS, pass_body, 0)

    return scatter_add


def kernel(inputs, indices, output_shape):
    Bs, Hp, Wp, Cs = inputs.shape
    H, W, C = Hp * 2, Wp * 2, Cs
    NB = Hp * Wp * Cs
    OUTB = H * W * C
    val_flat = inputs.reshape(-1)
    idx_flat = indices.reshape(-1).astype(jnp.int32)
    out = _build(Bs, NB, OUTB)(val_flat, idx_flat)
    return out.reshape(Bs, H, W, C)


# speedup vs baseline: 14.8573x; 1.2663x over previous
import functools

import jax
import jax.numpy as jnp
from jax import lax
from jax.experimental import pallas as pl
from jax.experimental.pallas import tpu as pltpu
from jax.experimental.pallas import tpu_sc as plsc

_NS = 16
_FB = 128
_UN = 8
_K = 1024


@functools.lru_cache(maxsize=None)
def _build(B, NB, OUTB):
    ET = NB // _NS
    assert ET % (2 * _K) == 0
    NCHUNK = ET // _K
    NPASS = 12
    assert OUTB % NPASS == 0
    W = OUTB // NPASS
    assert W % (_NS * 16) == 0
    PERT = W // _NS
    ZB = 3584
    assert PERT % ZB == 0
    CAP = _K + 2 * _FB

    mesh = plsc.VectorSubcoreMesh(core_axis_name="c", subcore_axis_name="s")

    @functools.partial(
        pl.kernel,
        out_type=jax.ShapeDtypeStruct((B * OUTB,), jnp.float32),
        mesh=mesh,
        compiler_params=pltpu.CompilerParams(needs_layout_passes=False),
        scratch_types=[
            pltpu.VMEM((_K,), jnp.int32),
            pltpu.VMEM((_K,), jnp.int32),
            pltpu.VMEM((_K,), jnp.float32),
            pltpu.VMEM((_K,), jnp.float32),
            pltpu.VMEM((CAP + 16,), jnp.int32),
            pltpu.VMEM((CAP + 16,), jnp.float32),
            pltpu.VMEM((ZB,), jnp.float32),
            pltpu.VMEM_SHARED((W,), jnp.float32),
            pltpu.SemaphoreType.DMA,
            pltpu.SemaphoreType.DMA,
        ],
    )
    def scatter_add(val_hbm, idx_hbm, out_hbm, idx0, idx1, val0, val1,
                    cidx, cval, zbuf, win, sem0, sem1):
        c = lax.axis_index("c")
        s = lax.axis_index("s")
        base_elem = c * NB + s * ET
        ibufs = (idx0, idx1)
        vbufs = (val0, val1)
        sems = (sem0, sem1)
        zeros_f = jnp.zeros((16,), jnp.float32)
        zeros_i = jnp.zeros((16,), jnp.int32)

        def zb_body(i, _):
            zbuf[pl.ds(i * 16, 16)] = zeros_f
            return 0
        lax.fori_loop(0, ZB // 16, zb_body, 0)

        for b in range(2):
            off = base_elem + b * _K
            pltpu.async_copy(idx_hbm.at[pl.ds(off, _K)], ibufs[b], sems[b])
            pltpu.async_copy(val_hbm.at[pl.ds(off, _K)], vbufs[b], sems[b])

        def pass_body(p, _):
            base = p * W

            def zw_body(z, _):
                pltpu.sync_copy(zbuf, win.at[pl.ds(s * PERT + z * ZB, ZB)])
                return 0
            lax.fori_loop(0, PERT // ZB, zw_body, 0)
            plsc.subcore_barrier()

            def pair_body(j2, fill):
                for b in range(2):
                    j = j2 * 2 + b
                    off = base_elem + j * _K
                    pltpu.make_async_copy(
                        idx_hbm.at[pl.ds(off, _K)], ibufs[b], sems[b]).wait()
                    pltpu.make_async_copy(
                        val_hbm.at[pl.ds(off, _K)], vbufs[b], sems[b]).wait()

                    def vec_body(i, fill):
                        dat = []
                        for q in range(_UN):
                            o = (i * _UN + q) * 16
                            iv = ibufs[b][pl.ds(o, 16)]
                            vv = vbufs[b][pl.ds(o, 16)]
                            t = iv - base
                            m = (t >= 0) & (t < W)
                            cnt = plsc.all_reduce_population_count(m)[0]
                            dat.append((m, t, vv, cnt))
                        for q in range(_UN):
                            m, t, vv, cnt = dat[q]
                            plsc.store_compressed(
                                cidx.at[pl.ds(fill, 16)], t, mask=m)
                            plsc.store_compressed(
                                cval.at[pl.ds(fill, 16)], vv, mask=m)
                            fill = fill + cnt
                        return fill
                    fill = lax.fori_loop(0, _K // (16 * _UN), vec_body, fill)

                    jn = j + 2
                    jn = jnp.where(jn >= NCHUNK, jn - NCHUNK, jn)
                    offn = base_elem + jn * _K
                    pltpu.async_copy(
                        idx_hbm.at[pl.ds(offn, _K)], ibufs[b], sems[b])
                    pltpu.async_copy(
                        val_hbm.at[pl.ds(offn, _K)], vbufs[b], sems[b])

                    nblk = fill // _FB

                    def fl_body(k, _):
                        pltpu.sync_copy(
                            cval.at[pl.ds(k * _FB, _FB)],
                            win.at[cidx.at[pl.ds(k * _FB, _FB)]],
                            add=True)
                        return 0
                    lax.fori_loop(0, nblk, fl_body, 0)

                    rem = fill - nblk * _FB
                    srcoff = nblk * _FB

                    def mv_body(i, _):
                        cidx[pl.ds(i * 16, 16)] = (
                            cidx[pl.ds(srcoff + i * 16, 16)])
                        cval[pl.ds(i * 16, 16)] = (
                            cval[pl.ds(srcoff + i * 16, 16)])
                        return 0
                    lax.fori_loop(0, (rem + 15) // 16, mv_body, 0)
                    fill = rem
                return fill
            fill = lax.fori_loop(0, NCHUNK // 2, pair_body, jnp.int32(0))

            def pad_body(i, _):
                cidx[pl.ds(fill + i * 16, 16)] = zeros_i
                cval[pl.ds(fill + i * 16, 16)] = zeros_f
                return 0
            lax.fori_loop(0, _FB // 16, pad_body, 0)
            nblk2 = (fill + _FB - 1) // _FB

            def fl2_body(k, _):
                pltpu.sync_copy(cval.at[pl.ds(k * _FB, _FB)],
                                win.at[cidx.at[pl.ds(k * _FB, _FB)]],
                                add=True)
                return 0
            lax.fori_loop(0, nblk2, fl2_body, 0)
            plsc.subcore_barrier()

            pltpu.sync_copy(
                win.at[pl.ds(s * PERT, PERT)],
                out_hbm.at[pl.ds(c * OUTB + base + s * PERT, PERT)])
            plsc.subcore_barrier()
            return 0
        lax.fori_loop(0, NPASS, pass_body, 0)

        for b in range(2):
            off = base_elem + b * _K
            pltpu.make_async_copy(
                idx_hbm.at[pl.ds(off, _K)], ibufs[b], sems[b]).wait()
            pltpu.make_async_copy(
                val_hbm.at[pl.ds(off, _K)], vbufs[b], sems[b]).wait()

    return scatter_add


def kernel(inputs, indices, output_shape):
    Bs, Hp, Wp, Cs = inputs.shape
    H, W, C = Hp * 2, Wp * 2, Cs
    NB = Hp * Wp * Cs
    OUTB = H * W * C
    val_flat = inputs.reshape(-1)
    idx_flat = indices.reshape(-1).astype(jnp.int32)
    out = _build(Bs, NB, OUTB)(val_flat, idx_flat)
    return out.reshape(Bs, H, W, C)


# speedup vs baseline: 14.9175x; 1.0041x over previous
import functools

import jax
import jax.numpy as jnp
from jax import lax
from jax.experimental import pallas as pl
from jax.experimental.pallas import tpu as pltpu
from jax.experimental.pallas import tpu_sc as plsc

_NS = 16
_FB = 128
_UN = 8
_K = 1024


@functools.lru_cache(maxsize=None)
def _build(B, NB, OUTB):
    ET = NB // _NS
    assert ET % (2 * _K) == 0
    NCHUNK = ET // _K
    NPASS = 12
    assert OUTB % NPASS == 0
    W = OUTB // NPASS
    assert W % (_NS * 16) == 0
    PERT = W // _NS
    ZB = 3584
    assert PERT % ZB == 0
    CAP = _K + 2 * _FB

    mesh = plsc.VectorSubcoreMesh(core_axis_name="c", subcore_axis_name="s")

    @functools.partial(
        pl.kernel,
        out_type=jax.ShapeDtypeStruct((B * OUTB,), jnp.float32),
        mesh=mesh,
        compiler_params=pltpu.CompilerParams(needs_layout_passes=False),
        scratch_types=[
            pltpu.VMEM((_K,), jnp.int32),
            pltpu.VMEM((_K,), jnp.int32),
            pltpu.VMEM((_K,), jnp.float32),
            pltpu.VMEM((_K,), jnp.float32),
            pltpu.VMEM((CAP + 16,), jnp.int32),
            pltpu.VMEM((CAP + 16,), jnp.int32),
            pltpu.VMEM((CAP + 16,), jnp.float32),
            pltpu.VMEM((CAP + 16,), jnp.float32),
            pltpu.VMEM((ZB,), jnp.float32),
            pltpu.VMEM_SHARED((W,), jnp.float32),
            pltpu.SemaphoreType.DMA,
            pltpu.SemaphoreType.DMA,
            pltpu.SemaphoreType.DMA,
            pltpu.SemaphoreType.DMA,
        ],
    )
    def scatter_add(val_hbm, idx_hbm, out_hbm, idx0, idx1, val0, val1,
                    ci0, ci1, cv0, cv1, zbuf, win, sem0, sem1, fsem0, fsem1):
        c = lax.axis_index("c")
        s = lax.axis_index("s")
        base_elem = c * NB + s * ET
        ibufs = (idx0, idx1)
        vbufs = (val0, val1)
        cibufs = (ci0, ci1)
        cvbufs = (cv0, cv1)
        sems = (sem0, sem1)
        fsems = (fsem0, fsem1)
        zeros_f = jnp.zeros((16,), jnp.float32)
        zeros_i = jnp.zeros((16,), jnp.int32)
        wlim = jnp.uint32(W)

        def zb_body(i, _):
            zbuf[pl.ds(i * 16, 16)] = zeros_f
            return 0
        lax.fori_loop(0, ZB // 16, zb_body, 0)

        for b in range(2):
            off = base_elem + b * _K
            pltpu.async_copy(idx_hbm.at[pl.ds(off, _K)], ibufs[b], sems[b])
            pltpu.async_copy(val_hbm.at[pl.ds(off, _K)], vbufs[b], sems[b])

        def pass_body(p, _):
            base = p * W

            def zw_body(z, _):
                pltpu.sync_copy(zbuf, win.at[pl.ds(s * PERT + z * ZB, ZB)])
                return 0
            lax.fori_loop(0, PERT // ZB, zw_body, 0)
            plsc.subcore_barrier()

            def pair_body(j2, carry):
                fill, out_prev = carry
                for b in range(2):
                    cidx, cval = cibufs[b], cvbufs[b]
                    j = j2 * 2 + b
                    off = base_elem + j * _K
                    pltpu.make_async_copy(
                        idx_hbm.at[pl.ds(off, _K)], ibufs[b], sems[b]).wait()
                    pltpu.make_async_copy(
                        val_hbm.at[pl.ds(off, _K)], vbufs[b], sems[b]).wait()

                    def vec_body(i, fill):
                        dat = []
                        for q in range(_UN):
                            o = (i * _UN + q) * 16
                            iv = ibufs[b][pl.ds(o, 16)]
                            vv = vbufs[b][pl.ds(o, 16)]
                            t = iv - base
                            tu = lax.bitcast_convert_type(t, jnp.uint32)
                            m = tu < wlim
                            cnt = plsc.all_reduce_population_count(m)[0]
                            dat.append((m, t, vv, cnt))
                        for q in range(_UN):
                            m, t, vv, cnt = dat[q]
                            plsc.store_compressed(
                                cidx.at[pl.ds(fill, 16)], t, mask=m)
                            plsc.store_compressed(
                                cval.at[pl.ds(fill, 16)], vv, mask=m)
                            fill = fill + cnt
                        return fill
                    fill = lax.fori_loop(0, _K // (16 * _UN), vec_body, fill)

                    jn = j + 2
                    jn = jnp.where(jn >= NCHUNK, jn - NCHUNK, jn)
                    offn = base_elem + jn * _K
                    pltpu.async_copy(
                        idx_hbm.at[pl.ds(offn, _K)], ibufs[b], sems[b])
                    pltpu.async_copy(
                        val_hbm.at[pl.ds(offn, _K)], vbufs[b], sems[b])

                    nblk = fill // _FB

                    def fl_body(k, _):
                        pltpu.async_copy(
                            cval.at[pl.ds(k * _FB, _FB)],
                            win.at[cidx.at[pl.ds(k * _FB, _FB)]],
                            fsems[b], add=True)
                        return 0
                    lax.fori_loop(0, nblk, fl_body, 0)

                    def dr_body(k, _):
                        pltpu.make_async_copy(
                            cvbufs[1 - b].at[pl.ds(0, _FB)],
                            win.at[cibufs[1 - b].at[pl.ds(0, _FB)]],
                            fsems[1 - b]).wait()
                        return 0
                    lax.fori_loop(0, out_prev, dr_body, 0)

                    rem = fill - nblk * _FB
                    srcoff = nblk * _FB

                    def mv_body(i, _):
                        cibufs[1 - b][pl.ds(i * 16, 16)] = (
                            cidx[pl.ds(srcoff + i * 16, 16)])
                        cvbufs[1 - b][pl.ds(i * 16, 16)] = (
                            cval[pl.ds(srcoff + i * 16, 16)])
                        return 0
                    lax.fori_loop(0, (rem + 15) // 16, mv_body, 0)
                    fill = rem
                    out_prev = nblk
                return fill, out_prev
            fill, out_prev = lax.fori_loop(
                0, NCHUNK // 2, pair_body, (jnp.int32(0), jnp.int32(0)))

            def pad_body(i, _):
                ci0[pl.ds(fill + i * 16, 16)] = zeros_i
                cv0[pl.ds(fill + i * 16, 16)] = zeros_f
                return 0
            lax.fori_loop(0, _FB // 16, pad_body, 0)
            nblk2 = (fill + _FB - 1) // _FB

            def fl2_body(k, _):
                pltpu.sync_copy(cv0.at[pl.ds(k * _FB, _FB)],
                                win.at[ci0.at[pl.ds(k * _FB, _FB)]],
                                add=True)
                return 0
            lax.fori_loop(0, nblk2, fl2_body, 0)

            def dr2_body(k, _):
                pltpu.make_async_copy(
                    cv1.at[pl.ds(0, _FB)],
                    win.at[ci1.at[pl.ds(0, _FB)]], fsem1).wait()
                return 0
            lax.fori_loop(0, out_prev, dr2_body, 0)
            plsc.subcore_barrier()

            pltpu.sync_copy(
                win.at[pl.ds(s * PERT, PERT)],
                out_hbm.at[pl.ds(c * OUTB + base + s * PERT, PERT)])
            plsc.subcore_barrier()
            return 0
        lax.fori_loop(0, NPASS, pass_body, 0)

        for b in range(2):
            off = base_elem + b * _K
            pltpu.make_async_copy(
                idx_hbm.at[pl.ds(off, _K)], ibufs[b], sems[b]).wait()
            pltpu.make_async_copy(
                val_hbm.at[pl.ds(off, _K)], vbufs[b], sems[b]).wait()

    return scatter_add


def kernel(inputs, indices, output_shape):
    Bs, Hp, Wp, Cs = inputs.shape
    H, W, C = Hp * 2, Wp * 2, Cs
    NB = Hp * Wp * Cs
    OUTB = H * W * C
    val_flat = inputs.reshape(-1)
    idx_flat = indices.reshape(-1).astype(jnp.int32)
    out = _build(Bs, NB, OUTB)(val_flat, idx_flat)
    return out.reshape(Bs, H, W, C)


# speedup vs baseline: 23.9911x; 1.6082x over previous
import functools

import jax
import jax.numpy as jnp
from jax import lax
from jax.experimental import pallas as pl
from jax.experimental.pallas import tpu as pltpu
from jax.experimental.pallas import tpu_sc as plsc

_NS = 16
_FB = 128
_UN = 16
_K = 3072


@functools.lru_cache(maxsize=None)
def _build(B, NB, OUTB):
    ET = NB // _NS
    assert ET % (2 * _K) == 0
    NCHUNK = ET // _K
    NPASS = 12
    assert OUTB % NPASS == 0
    W = OUTB // NPASS
    assert W % (_NS * 16) == 0
    PERT = W // _NS
    ZB = 3584
    assert PERT % ZB == 0
    CAP = _K + 2 * _FB

    mesh = plsc.VectorSubcoreMesh(core_axis_name="c", subcore_axis_name="s")

    @functools.partial(
        pl.kernel,
        out_type=jax.ShapeDtypeStruct((B * OUTB,), jnp.float32),
        mesh=mesh,
        compiler_params=pltpu.CompilerParams(needs_layout_passes=False),
        scratch_types=[
            pltpu.VMEM((_K,), jnp.int32),
            pltpu.VMEM((_K,), jnp.int32),
            pltpu.VMEM((_K,), jnp.float32),
            pltpu.VMEM((_K,), jnp.float32),
            pltpu.VMEM((CAP + 16,), jnp.int32),
            pltpu.VMEM((CAP + 16,), jnp.int32),
            pltpu.VMEM((CAP + 16,), jnp.float32),
            pltpu.VMEM((CAP + 16,), jnp.float32),
            pltpu.VMEM((ZB,), jnp.float32),
            pltpu.VMEM_SHARED((W,), jnp.float32),
            pltpu.SemaphoreType.DMA,
            pltpu.SemaphoreType.DMA,
            pltpu.SemaphoreType.DMA,
            pltpu.SemaphoreType.DMA,
        ],
    )
    def scatter_add(val_hbm, idx_hbm, out_hbm, idx0, idx1, val0, val1,
                    ci0, ci1, cv0, cv1, zbuf, win, sem0, sem1, fsem0, fsem1):
        c = lax.axis_index("c")
        s = lax.axis_index("s")
        base_elem = c * NB + s * ET
        ibufs = (idx0, idx1)
        vbufs = (val0, val1)
        cibufs = (ci0, ci1)
        cvbufs = (cv0, cv1)
        sems = (sem0, sem1)
        fsems = (fsem0, fsem1)
        zeros_f = jnp.zeros((16,), jnp.float32)
        zeros_i = jnp.zeros((16,), jnp.int32)
        wlim = jnp.uint32(W)

        def zb_body(i, _):
            zbuf[pl.ds(i * 16, 16)] = zeros_f
            return 0
        lax.fori_loop(0, ZB // 16, zb_body, 0)

        for b in range(2):
            off = base_elem + b * _K
            pltpu.async_copy(idx_hbm.at[pl.ds(off, _K)], ibufs[b], sems[b])
            pltpu.async_copy(val_hbm.at[pl.ds(off, _K)], vbufs[b], sems[b])

        def pass_body(p, _):
            base = p * W

            def zw_body(z, _):
                pltpu.sync_copy(zbuf, win.at[pl.ds(s * PERT + z * ZB, ZB)])
                return 0
            lax.fori_loop(0, PERT // ZB, zw_body, 0)
            plsc.subcore_barrier()

            def pair_body(j2, carry):
                fill, out_prev = carry
                for b in range(2):
                    cidx, cval = cibufs[b], cvbufs[b]
                    j = j2 * 2 + b
                    off = base_elem + j * _K
                    pltpu.make_async_copy(
                        idx_hbm.at[pl.ds(off, _K)], ibufs[b], sems[b]).wait()
                    pltpu.make_async_copy(
                        val_hbm.at[pl.ds(off, _K)], vbufs[b], sems[b]).wait()

                    def vec_body(i, fill):
                        dat = []
                        for q in range(_UN):
                            o = (i * _UN + q) * 16
                            iv = ibufs[b][pl.ds(o, 16)]
                            vv = vbufs[b][pl.ds(o, 16)]
                            t = iv - base
                            tu = lax.bitcast_convert_type(t, jnp.uint32)
                            m = tu < wlim
                            cnt = plsc.all_reduce_population_count(m)[0]
                            dat.append((m, t, vv, cnt))
                        for q in range(_UN):
                            m, t, vv, cnt = dat[q]
                            plsc.store_compressed(
                                cidx.at[pl.ds(fill, 16)], t, mask=m)
                            plsc.store_compressed(
                                cval.at[pl.ds(fill, 16)], vv, mask=m)
                            fill = fill + cnt
                        return fill
                    fill = lax.fori_loop(0, _K // (16 * _UN), vec_body, fill)

                    jn = j + 2
                    jn = jnp.where(jn >= NCHUNK, jn - NCHUNK, jn)
                    offn = base_elem + jn * _K
                    pltpu.async_copy(
                        idx_hbm.at[pl.ds(offn, _K)], ibufs[b], sems[b])
                    pltpu.async_copy(
                        val_hbm.at[pl.ds(offn, _K)], vbufs[b], sems[b])

                    nblk = fill // _FB

                    def fl_body(k, _):
                        pltpu.async_copy(
                            cval.at[pl.ds(k * _FB, _FB)],
                            win.at[cidx.at[pl.ds(k * _FB, _FB)]],
                            fsems[b], add=True)
                        return 0
                    lax.fori_loop(0, nblk, fl_body, 0)

                    def dr_body(k, _):
                        pltpu.make_async_copy(
                            cvbufs[1 - b].at[pl.ds(0, _FB)],
                            win.at[cibufs[1 - b].at[pl.ds(0, _FB)]],
                            fsems[1 - b]).wait()
                        return 0
                    lax.fori_loop(0, out_prev, dr_body, 0)

                    rem = fill - nblk * _FB
                    srcoff = nblk * _FB

                    def mv_body(i, _):
                        cibufs[1 - b][pl.ds(i * 16, 16)] = (
                            cidx[pl.ds(srcoff + i * 16, 16)])
                        cvbufs[1 - b][pl.ds(i * 16, 16)] = (
                            cval[pl.ds(srcoff + i * 16, 16)])
                        return 0
                    lax.fori_loop(0, (rem + 15) // 16, mv_body, 0)
                    fill = rem
                    out_prev = nblk
                return fill, out_prev
            fill, out_prev = lax.fori_loop(
                0, NCHUNK // 2, pair_body, (jnp.int32(0), jnp.int32(0)))

            def pad_body(i, _):
                ci0[pl.ds(fill + i * 16, 16)] = zeros_i
                cv0[pl.ds(fill + i * 16, 16)] = zeros_f
                return 0
            lax.fori_loop(0, _FB // 16, pad_body, 0)
            nblk2 = (fill + _FB - 1) // _FB

            def fl2_body(k, _):
                pltpu.sync_copy(cv0.at[pl.ds(k * _FB, _FB)],
                                win.at[ci0.at[pl.ds(k * _FB, _FB)]],
                                add=True)
                return 0
            lax.fori_loop(0, nblk2, fl2_body, 0)

            def dr2_body(k, _):
                pltpu.make_async_copy(
                    cv1.at[pl.ds(0, _FB)],
                    win.at[ci1.at[pl.ds(0, _FB)]], fsem1).wait()
                return 0
            lax.fori_loop(0, out_prev, dr2_body, 0)
            plsc.subcore_barrier()

            pltpu.sync_copy(
                win.at[pl.ds(s * PERT, PERT)],
                out_hbm.at[pl.ds(c * OUTB + base + s * PERT, PERT)])
            plsc.subcore_barrier()
            return 0
        lax.fori_loop(0, NPASS, pass_body, 0)

        for b in range(2):
            off = base_elem + b * _K
            pltpu.make_async_copy(
                idx_hbm.at[pl.ds(off, _K)], ibufs[b], sems[b]).wait()
            pltpu.make_async_copy(
                val_hbm.at[pl.ds(off, _K)], vbufs[b], sems[b]).wait()

    return scatter_add


def kernel(inputs, indices, output_shape):
    Bs, Hp, Wp, Cs = inputs.shape
    H, W, C = Hp * 2, Wp * 2, Cs
    NB = Hp * Wp * Cs
    OUTB = H * W * C
    val_flat = inputs.reshape(-1)
    idx_flat = indices.reshape(-1).astype(jnp.int32)
    out = _build(Bs, NB, OUTB)(val_flat, idx_flat)
    return out.reshape(Bs, H, W, C)
